# baseline (speedup 1.0000x reference)
import jax
import jax.numpy as jnp
from jax.experimental import pallas as pl
from jax.experimental.pallas import tpu as pltpu

_CHUNKS = (1536, 4608, 6144)
_OFFS = (0, 1536, 6144)


def _copy_kernel(in_ref, out_ref, b0, b1, b2, in_sems, out_sems):
    bufs = (b0, b1, b2)
    for i, (off, n) in enumerate(zip(_OFFS, _CHUNKS)):
        pltpu.make_async_copy(
            in_ref.at[pl.ds(off, n)], bufs[i], in_sems.at[i]
        ).start()
    for i, (off, n) in enumerate(zip(_OFFS, _CHUNKS)):
        pltpu.make_async_copy(
            in_ref.at[pl.ds(off, n)], bufs[i], in_sems.at[i]
        ).wait()
        pltpu.make_async_copy(
            bufs[i], out_ref.at[pl.ds(off, n)], out_sems.at[i]
        ).start()
    for i, (off, n) in enumerate(zip(_OFFS, _CHUNKS)):
        pltpu.make_async_copy(
            bufs[i], out_ref.at[pl.ds(off, n)], out_sems.at[i]
        ).wait()


def kernel(images):
    flat = images.reshape(-1, 512)
    out = pl.pallas_call(
        _copy_kernel,
        out_shape=jax.ShapeDtypeStruct(flat.shape, flat.dtype),
        in_specs=[pl.BlockSpec(memory_space=pl.ANY)],
        out_specs=pl.BlockSpec(memory_space=pl.ANY),
        scratch_shapes=[
            pltpu.VMEM((_CHUNKS[0], 512), jnp.float32),
            pltpu.VMEM((_CHUNKS[1], 512), jnp.float32),
            pltpu.VMEM((_CHUNKS[2], 512), jnp.float32),
            pltpu.SemaphoreType.DMA((3,)),
            pltpu.SemaphoreType.DMA((3,)),
        ],
    )(flat)
    return out.reshape(images.shape)
